# Initial kernel scaffold; baseline (speedup 1.0000x reference)
import jax
import jax.numpy as jnp
from jax.experimental import pallas as pl


def kernel(x, edge_index, W1_0, b1_0, W2_0, b2_0, W1_1, b1_1, W2_1, b2_1):
    raise NotImplementedError("write your pallas kernel here")


# speedup vs baseline: 3.1964x; 3.1964x over previous
import functools

import jax
import jax.numpy as jnp
from jax import lax
from jax.experimental import pallas as pl
from jax.experimental.pallas import tpu as pltpu
from jax.experimental.pallas import tpu_sc as plsc

_NC = 2
_NS = 16
_NW = _NC * _NS
_G = 128


@functools.lru_cache(maxsize=None)
def _make_gather(N, R, H):
    mesh = plsc.VectorSubcoreMesh(core_axis_name="c", subcore_axis_name="s")
    per_w = (R + _NW - 1) // _NW

    @functools.partial(
        pl.kernel,
        out_type=jax.ShapeDtypeStruct((R * _G, H), jnp.float32),
        mesh=mesh,
        scratch_types=[
            pltpu.VMEM((1, _G), jnp.int32),
            pltpu.VMEM((1, _G), jnp.int32),
            pltpu.VMEM((_G, H), jnp.float32),
            pltpu.VMEM((_G, H), jnp.float32),
            pltpu.SemaphoreType.DMA,
        ],
        compiler_params=pltpu.CompilerParams(use_tc_tiling_on_sc=False),
    )
    def gather_k(a_hbm, b_hbm, dst_hbm, src_hbm, h_hbm, idxd, idxs, abuf,
                 bbuf, sem):
        c = lax.axis_index("c")
        s = lax.axis_index("s")
        wid = s * _NC + c
        r0 = wid * per_w

        def body(i, carry):
            row = r0 + i

            @pl.when(row < R)
            def _():
                pltpu.sync_copy(dst_hbm.at[pl.ds(row, 1)], idxd)
                pltpu.sync_copy(src_hbm.at[pl.ds(row, 1)], idxs)
                cp1 = pltpu.async_copy(a_hbm.at[idxd.at[0]], abuf, sem)
                cp2 = pltpu.async_copy(b_hbm.at[idxs.at[0]], bbuf, sem)
                cp1.wait()
                cp2.wait()

                def vbody(r, carry2):
                    for j in range(H // 16):
                        sl = pl.ds(j * 16, 16)
                        abuf[r, sl] = jnp.maximum(abuf[r, sl] + bbuf[r, sl],
                                                  0.0)
                    return carry2

                lax.fori_loop(0, _G, vbody, 0)
                pltpu.sync_copy(abuf, h_hbm.at[pl.ds(row * _G, _G)])

            return carry

        lax.fori_loop(0, per_w, body, 0)

    return gather_k


@functools.lru_cache(maxsize=None)
def _make_scatter(N_pad, R, C, with_counts):
    mesh = plsc.VectorSubcoreMesh(core_axis_name="c", subcore_axis_name="s")
    r_core = R // _NC
    per_s = (r_core + _NS - 1) // _NS
    n_sub = N_pad // _NS
    zrows = 128
    nz = n_sub // zrows

    out_type = [jax.ShapeDtypeStruct((_NC, N_pad, C), jnp.float32)]
    scratch = [
        pltpu.VMEM((1, _G), jnp.int32),
        pltpu.VMEM((_G, C), jnp.float32),
        pltpu.VMEM((zrows, C), jnp.float32),
        pltpu.VMEM_SHARED((N_pad, C), jnp.float32),
        pltpu.SemaphoreType.DMA,
    ]
    if with_counts:
        out_type.append(jax.ShapeDtypeStruct((_NC, N_pad, 16), jnp.float32))
        scratch += [
            pltpu.VMEM((_G, 16), jnp.float32),
            pltpu.VMEM((zrows, 16), jnp.float32),
            pltpu.VMEM_SHARED((N_pad, 16), jnp.float32),
        ]

    @functools.partial(pl.kernel, out_type=out_type, mesh=mesh,
                       scratch_types=scratch,
                       compiler_params=pltpu.CompilerParams(
                           use_tc_tiling_on_sc=False))
    def scatter_k(m_hbm, dst_hbm, *refs):
        if with_counts:
            (s_out, c_out, idxd, mbuf, zbuf, s_sh, sem, ones, zbufc,
             c_sh) = refs
        else:
            s_out, idxd, mbuf, zbuf, s_sh, sem = refs
        c = lax.axis_index("c")
        s = lax.axis_index("s")

        def zbody(i, carry):
            for j in range(C // 16):
                zbuf[i, pl.ds(j * 16, 16)] = jnp.zeros((16,), jnp.float32)
            if with_counts:
                zbufc[i, pl.ds(0, 16)] = jnp.zeros((16,), jnp.float32)
            return carry

        lax.fori_loop(0, zrows, zbody, 0)
        if with_counts:
            def obody(i, carry):
                lane = lax.iota(jnp.int32, 16)
                ones[i, pl.ds(0, 16)] = jnp.where(lane == 0, 1.0, 0.0)
                return carry

            lax.fori_loop(0, _G, obody, 0)
        for k in range(nz):
            dst_sl = pl.ds(s * n_sub + k * zrows, zrows)
            pltpu.sync_copy(zbuf, s_sh.at[dst_sl])
            if with_counts:
                pltpu.sync_copy(zbufc, c_sh.at[dst_sl])
        plsc.subcore_barrier()

        r0 = c * r_core + s * per_s
        rend = (c + 1) * r_core

        def body(i, carry):
            row = r0 + i

            @pl.when(row < rend)
            def _():
                pltpu.sync_copy(dst_hbm.at[pl.ds(row, 1)], idxd)
                pltpu.sync_copy(m_hbm.at[pl.ds(row * _G, _G)], mbuf)
                pltpu.sync_copy(mbuf, s_sh.at[idxd.at[0]], add=True)
                if with_counts:
                    pltpu.sync_copy(ones, c_sh.at[idxd.at[0]], add=True)

            return carry

        lax.fori_loop(0, per_s, body, 0)
        plsc.subcore_barrier()

        my = pl.ds(s * n_sub, n_sub)
        pltpu.sync_copy(s_sh.at[my], s_out.at[c, my])
        if with_counts:
            pltpu.sync_copy(c_sh.at[my], c_out.at[c, my])

    return scatter_k


def _dense_pre(x, W1, b1):
    N, D = x.shape
    H = W1.shape[1]

    def body(x_ref, wt_ref, wb_ref, b1_ref, a_ref, b_ref):
        xv = x_ref[...]
        a_ref[...] = (jnp.dot(xv, wt_ref[...],
                              preferred_element_type=jnp.float32)
                      + b1_ref[...])
        b_ref[...] = jnp.dot(xv, wb_ref[...],
                             preferred_element_type=jnp.float32)

    return pl.pallas_call(
        body,
        out_shape=(jax.ShapeDtypeStruct((N, H), jnp.float32),
                   jax.ShapeDtypeStruct((N, H), jnp.float32)),
    )(x, W1[:D], W1[D:], b1.reshape(1, H))


def _combine_pre(S, Cnt, W1, b1):
    _, N, D = S.shape
    H = W1.shape[1]

    def body(s_ref, c_ref, wt_ref, wb_ref, b1_ref, a_ref, b_ref):
        cnt = c_ref[0, :, 0] + c_ref[1, :, 0]
        xv = (s_ref[0] + s_ref[1]) / jnp.maximum(cnt, 1.0)[:, None]
        a_ref[...] = (jnp.dot(xv, wt_ref[...],
                              preferred_element_type=jnp.float32)
                      + b1_ref[...])
        b_ref[...] = jnp.dot(xv, wb_ref[...],
                             preferred_element_type=jnp.float32)

    return pl.pallas_call(
        body,
        out_shape=(jax.ShapeDtypeStruct((N, H), jnp.float32),
                   jax.ShapeDtypeStruct((N, H), jnp.float32)),
    )(S, Cnt, W1[:D], W1[D:], b1.reshape(1, H))


def _edge_mlp(Hm, W2, b2):
    E, H = Hm.shape
    C = W2.shape[1]
    BE = 2560
    assert E % BE == 0

    def body(h_ref, w_ref, b_ref, m_ref):
        m_ref[...] = jnp.maximum(
            jnp.dot(h_ref[...], w_ref[...],
                    preferred_element_type=jnp.float32) + b_ref[...], 0.0)

    return pl.pallas_call(
        body,
        grid=(E // BE,),
        in_specs=[
            pl.BlockSpec((BE, H), lambda i: (i, 0)),
            pl.BlockSpec((H, C), lambda i: (0, 0)),
            pl.BlockSpec((1, C), lambda i: (0, 0)),
        ],
        out_specs=pl.BlockSpec((BE, C), lambda i: (i, 0)),
        out_shape=jax.ShapeDtypeStruct((E, C), jnp.float32),
    )(Hm, W2, b2.reshape(1, C))


def _final_combine(S, Cnt, N):
    _, _, C = S.shape

    def body(s_ref, c_ref, o_ref):
        cnt = c_ref[0, :N, 0] + c_ref[1, :N, 0]
        o_ref[...] = (s_ref[0, :N] + s_ref[1, :N]) / jnp.maximum(
            cnt, 1.0)[:, None]

    return pl.pallas_call(
        body,
        out_shape=jax.ShapeDtypeStruct((N, C), jnp.float32),
    )(S, Cnt)


def kernel(x, edge_index, W1_0, b1_0, W2_0, b2_0, W1_1, b1_1, W2_1, b2_1):
    N, D = x.shape
    E = edge_index.shape[1]
    H = W1_0.shape[1]
    assert E % _G == 0
    R = E // _G
    chunk = _NS * 128
    N_pad = ((N + chunk - 1) // chunk) * chunk

    dst2 = edge_index[0].reshape(R, _G)
    src2 = edge_index[1].reshape(R, _G)

    A0, B0 = _dense_pre(x, W1_0, b1_0)
    Hm0 = _make_gather(N, R, H)(A0, B0, dst2, src2)
    M0 = _edge_mlp(Hm0, W2_0, b2_0)
    S0, Cnt = _make_scatter(N_pad, R, W2_0.shape[1], True)(M0, dst2)

    A1, B1 = _combine_pre(S0, Cnt, W1_1, b1_1)
    Hm1 = _make_gather(N, R, H)(A1, B1, dst2, src2)
    M1 = _edge_mlp(Hm1, W2_1, b2_1)
    (S1,) = _make_scatter(N_pad, R, W2_1.shape[1], False)(M1, dst2)

    return _final_combine(S1, Cnt, N)


# speedup vs baseline: 3.3313x; 1.0422x over previous
import functools

import jax
import jax.numpy as jnp
from jax import lax
from jax.experimental import pallas as pl
from jax.experimental.pallas import tpu as pltpu
from jax.experimental.pallas import tpu_sc as plsc

_NC = 2
_NS = 16
_NW = _NC * _NS
_G = 128

_SC_PARAMS = pltpu.CompilerParams(use_tc_tiling_on_sc=False)


@functools.lru_cache(maxsize=None)
def _make_gather(N, R, H):
    mesh = plsc.VectorSubcoreMesh(core_axis_name="c", subcore_axis_name="s")
    per_w = (R + _NW - 1) // _NW
    n_stage = N // _NS

    @functools.partial(
        pl.kernel,
        out_type=jax.ShapeDtypeStruct((R * _G, H), jnp.float32),
        mesh=mesh,
        scratch_types=[
            pltpu.VMEM((2, 1, _G), jnp.int32),
            pltpu.VMEM((2, 1, _G), jnp.int32),
            pltpu.VMEM((2, _G, H), jnp.float32),
            pltpu.VMEM((2, _G, H), jnp.float32),
            pltpu.VMEM_SHARED((N, H), jnp.float32),
            pltpu.VMEM_SHARED((N, H), jnp.float32),
            pltpu.SemaphoreType.DMA,
            pltpu.SemaphoreType.DMA,
            pltpu.SemaphoreType.DMA,
        ],
        compiler_params=_SC_PARAMS,
    )
    def gather_k(a_hbm, b_hbm, dst_hbm, src_hbm, h_hbm, idxd, idxs, abuf,
                 bbuf, a_sh, b_sh, sem_idx, sem_g, sem_w):
        c = lax.axis_index("c")
        s = lax.axis_index("s")
        wid = s * _NC + c
        r0 = wid * per_w

        st = pl.ds(s * n_stage, n_stage)
        pltpu.sync_copy(a_hbm.at[st], a_sh.at[st])
        pltpu.sync_copy(b_hbm.at[st], b_sh.at[st])
        plsc.subcore_barrier()

        def issue_idx(row, slot):
            pltpu.async_copy(dst_hbm.at[pl.ds(row, 1)], idxd.at[slot],
                             sem_idx)
            pltpu.async_copy(src_hbm.at[pl.ds(row, 1)], idxs.at[slot],
                             sem_idx)

        def wait_idx(slot):
            pltpu.make_async_copy(dst_hbm.at[pl.ds(0, 1)], idxd.at[slot],
                                  sem_idx).wait()
            pltpu.make_async_copy(src_hbm.at[pl.ds(0, 1)], idxs.at[slot],
                                  sem_idx).wait()

        def wait_write(slot):
            pltpu.make_async_copy(abuf.at[slot], h_hbm.at[pl.ds(0, _G)],
                                  sem_w).wait()

        @pl.when(r0 < R)
        def _():
            issue_idx(r0, 0)

        def body(i, carry):
            row = r0 + i
            p = jnp.bitwise_and(i, 1)

            @pl.when((i >= 2) & (row - 2 < R))
            def _():
                wait_write(p)

            @pl.when(row < R)
            def _():
                wait_idx(p)
                ga = pltpu.async_copy(a_sh.at[idxd.at[p, 0]], abuf.at[p],
                                      sem_g)
                gb = pltpu.async_copy(b_sh.at[idxs.at[p, 0]], bbuf.at[p],
                                      sem_g)

                @pl.when((i < per_w - 1) & (row + 1 < R))
                def _():
                    issue_idx(row + 1, 1 - p)

                ga.wait()
                gb.wait()

                def vbody(r, carry2):
                    for j in range(H // 16):
                        sl = pl.ds(j * 16, 16)
                        abuf[p, r, sl] = jnp.maximum(
                            abuf[p, r, sl] + bbuf[p, r, sl], 0.0)
                    return carry2

                lax.fori_loop(0, _G, vbody, 0)
                pltpu.async_copy(abuf.at[p], h_hbm.at[pl.ds(row * _G, _G)],
                                 sem_w)

            return carry

        lax.fori_loop(0, per_w, body, 0)

        nv = jnp.clip(R - r0, 0, per_w)

        @pl.when(nv >= per_w - 1)
        def _():
            wait_write(0)

        @pl.when(nv >= per_w)
        def _():
            wait_write(0)

    return gather_k


@functools.lru_cache(maxsize=None)
def _make_scatter(N, R, C, with_counts):
    mesh = plsc.VectorSubcoreMesh(core_axis_name="c", subcore_axis_name="s")
    r_core = R // _NC
    per_s = (r_core + _NS - 1) // _NS
    n_sub = N // _NS
    zrows = 125
    nz = n_sub // zrows
    assert n_sub % zrows == 0

    out_type = [jax.ShapeDtypeStruct((_NC, N, C), jnp.float32)]
    scratch = [
        pltpu.VMEM((2, 1, _G), jnp.int32),
        pltpu.VMEM((2, _G, C), jnp.float32),
        pltpu.VMEM_SHARED((N, C), jnp.float32),
        pltpu.SemaphoreType.DMA,
        pltpu.SemaphoreType.DMA,
    ]
    if with_counts:
        out_type.append(jax.ShapeDtypeStruct((_NC, N, 16), jnp.float32))
        scratch += [
            pltpu.VMEM((_G, 16), jnp.float32),
            pltpu.VMEM_SHARED((N, 16), jnp.float32),
        ]

    @functools.partial(pl.kernel, out_type=out_type, mesh=mesh,
                       scratch_types=scratch, compiler_params=_SC_PARAMS)
    def scatter_k(m_hbm, dst_hbm, zs_hbm, zc_hbm, *refs):
        if with_counts:
            (s_out, c_out, idxd, mbuf, s_sh, sem_idx, sem_m, ones,
             c_sh) = refs
        else:
            s_out, idxd, mbuf, s_sh, sem_idx, sem_m = refs
        c = lax.axis_index("c")
        s = lax.axis_index("s")

        if with_counts:
            def obody(i, carry):
                lane = lax.iota(jnp.int32, 16)
                ones[i, pl.ds(0, 16)] = jnp.where(lane == 0, 1.0, 0.0)
                return carry

            lax.fori_loop(0, _G, obody, 0)
        for k in range(nz):
            dst_sl = pl.ds(s * n_sub + k * zrows, zrows)
            pltpu.sync_copy(zs_hbm, s_sh.at[dst_sl])
            if with_counts:
                pltpu.sync_copy(zc_hbm, c_sh.at[dst_sl])
        plsc.subcore_barrier()

        r0 = c * r_core + s * per_s
        rend = (c + 1) * r_core

        def issue(row, slot):
            pltpu.async_copy(dst_hbm.at[pl.ds(row, 1)], idxd.at[slot],
                             sem_idx)
            pltpu.async_copy(m_hbm.at[pl.ds(row * _G, _G)], mbuf.at[slot],
                             sem_m)

        def wait_in(slot):
            pltpu.make_async_copy(dst_hbm.at[pl.ds(0, 1)], idxd.at[slot],
                                  sem_idx).wait()
            pltpu.make_async_copy(m_hbm.at[pl.ds(0, _G)], mbuf.at[slot],
                                  sem_m).wait()

        @pl.when(r0 < rend)
        def _():
            issue(r0, 0)

        def body(i, carry):
            row = r0 + i
            p = jnp.bitwise_and(i, 1)

            @pl.when(row < rend)
            def _():
                wait_in(p)

                @pl.when((i < per_s - 1) & (row + 1 < rend))
                def _():
                    issue(row + 1, 1 - p)

                pltpu.sync_copy(mbuf.at[p], s_sh.at[idxd.at[p, 0]],
                                add=True)
                if with_counts:
                    pltpu.sync_copy(ones, c_sh.at[idxd.at[p, 0]], add=True)

            return carry

        lax.fori_loop(0, per_s, body, 0)
        plsc.subcore_barrier()

        my = pl.ds(s * n_sub, n_sub)
        pltpu.sync_copy(s_sh.at[my], s_out.at[c, my])
        if with_counts:
            pltpu.sync_copy(c_sh.at[my], c_out.at[c, my])

    return scatter_k


def _dense_pre(x, W1, b1):
    N, D = x.shape
    H = W1.shape[1]

    def body(x_ref, wt_ref, wb_ref, b1_ref, a_ref, b_ref):
        xv = x_ref[...]
        a_ref[...] = (jnp.dot(xv, wt_ref[...],
                              preferred_element_type=jnp.float32)
                      + b1_ref[...])
        b_ref[...] = jnp.dot(xv, wb_ref[...],
                             preferred_element_type=jnp.float32)

    return pl.pallas_call(
        body,
        out_shape=(jax.ShapeDtypeStruct((N, H), jnp.float32),
                   jax.ShapeDtypeStruct((N, H), jnp.float32)),
    )(x, W1[:D], W1[D:], b1.reshape(1, H))


def _combine_pre(S, Cnt, W1, b1):
    _, N, D = S.shape
    H = W1.shape[1]

    def body(s_ref, c_ref, wt_ref, wb_ref, b1_ref, a_ref, b_ref):
        cnt = c_ref[0, :, 0] + c_ref[1, :, 0]
        xv = (s_ref[0] + s_ref[1]) / jnp.maximum(cnt, 1.0)[:, None]
        a_ref[...] = (jnp.dot(xv, wt_ref[...],
                              preferred_element_type=jnp.float32)
                      + b1_ref[...])
        b_ref[...] = jnp.dot(xv, wb_ref[...],
                             preferred_element_type=jnp.float32)

    return pl.pallas_call(
        body,
        out_shape=(jax.ShapeDtypeStruct((N, H), jnp.float32),
                   jax.ShapeDtypeStruct((N, H), jnp.float32)),
    )(S, Cnt, W1[:D], W1[D:], b1.reshape(1, H))


def _edge_mlp(Hm, W2, b2):
    E, H = Hm.shape
    C = W2.shape[1]
    BE = 2560
    assert E % BE == 0

    def body(h_ref, w_ref, b_ref, m_ref):
        m_ref[...] = jnp.maximum(
            jnp.dot(h_ref[...], w_ref[...],
                    preferred_element_type=jnp.float32) + b_ref[...], 0.0)

    return pl.pallas_call(
        body,
        grid=(E // BE,),
        in_specs=[
            pl.BlockSpec((BE, H), lambda i: (i, 0)),
            pl.BlockSpec((H, C), lambda i: (0, 0)),
            pl.BlockSpec((1, C), lambda i: (0, 0)),
        ],
        out_specs=pl.BlockSpec((BE, C), lambda i: (i, 0)),
        out_shape=jax.ShapeDtypeStruct((E, C), jnp.float32),
    )(Hm, W2, b2.reshape(1, C))


def _final_combine(S, Cnt, N):
    _, _, C = S.shape

    def body(s_ref, c_ref, o_ref):
        cnt = c_ref[0, :N, 0] + c_ref[1, :N, 0]
        o_ref[...] = (s_ref[0, :N] + s_ref[1, :N]) / jnp.maximum(
            cnt, 1.0)[:, None]

    return pl.pallas_call(
        body,
        out_shape=jax.ShapeDtypeStruct((N, C), jnp.float32),
    )(S, Cnt)


def kernel(x, edge_index, W1_0, b1_0, W2_0, b2_0, W1_1, b1_1, W2_1, b2_1):
    N, D = x.shape
    E = edge_index.shape[1]
    H = W1_0.shape[1]
    assert E % _G == 0
    R = E // _G

    dst2 = edge_index[0].reshape(R, _G)
    src2 = edge_index[1].reshape(R, _G)

    A0, B0 = _dense_pre(x, W1_0, b1_0)
    Hm0 = _make_gather(N, R, H)(A0, B0, dst2, src2)
    M0 = _edge_mlp(Hm0, W2_0, b2_0)
    zs = jnp.zeros((125, W2_0.shape[1]), jnp.float32)
    zc = jnp.zeros((125, 16), jnp.float32)
    S0, Cnt = _make_scatter(N, R, W2_0.shape[1], True)(M0, dst2, zs, zc)

    A1, B1 = _combine_pre(S0, Cnt, W1_1, b1_1)
    Hm1 = _make_gather(N, R, H)(A1, B1, dst2, src2)
    M1 = _edge_mlp(Hm1, W2_1, b2_1)
    (S1,) = _make_scatter(N, R, W2_1.shape[1], False)(M1, dst2, zs, zc)

    return _final_combine(S1, Cnt, N)


# speedup vs baseline: 3.4762x; 1.0435x over previous
import functools

import jax
import jax.numpy as jnp
from jax import lax
from jax.experimental import pallas as pl
from jax.experimental.pallas import tpu as pltpu
from jax.experimental.pallas import tpu_sc as plsc

_NC = 2
_NS = 16
_NW = _NC * _NS
_G = 128

_SC_PARAMS = pltpu.CompilerParams(use_tc_tiling_on_sc=False)


@functools.lru_cache(maxsize=None)
def _make_gather(N, R, H):
    mesh = plsc.VectorSubcoreMesh(core_axis_name="c", subcore_axis_name="s")
    per_w = (R + _NW - 1) // _NW

    @functools.partial(
        pl.kernel,
        out_type=jax.ShapeDtypeStruct((R * _G, H), jnp.float32),
        mesh=mesh,
        scratch_types=[
            pltpu.VMEM((2, 1, _G), jnp.int32),
            pltpu.VMEM((2, 1, _G), jnp.int32),
            pltpu.VMEM((2, _G, H), jnp.float32),
            pltpu.VMEM((2, _G, H), jnp.float32),
            pltpu.SemaphoreType.DMA,
            pltpu.SemaphoreType.DMA,
            pltpu.SemaphoreType.DMA,
        ],
        compiler_params=_SC_PARAMS,
    )
    def gather_k(a_hbm, b_hbm, dst_hbm, src_hbm, h_hbm, idxd, idxs, abuf,
                 bbuf, sem_idx, sem_g, sem_w):
        c = lax.axis_index("c")
        s = lax.axis_index("s")
        wid = s * _NC + c
        r0 = wid * per_w
        nv = jnp.clip(R - r0, 0, per_w)

        def issue_idx(row, slot):
            pltpu.async_copy(dst_hbm.at[pl.ds(row, 1)], idxd.at[slot],
                             sem_idx)
            pltpu.async_copy(src_hbm.at[pl.ds(row, 1)], idxs.at[slot],
                             sem_idx)

        def wait_idx(slot):
            pltpu.make_async_copy(dst_hbm.at[pl.ds(0, 1)], idxd.at[slot],
                                  sem_idx).wait()
            pltpu.make_async_copy(src_hbm.at[pl.ds(0, 1)], idxs.at[slot],
                                  sem_idx).wait()

        def issue_gather(slot):
            pltpu.async_copy(a_hbm.at[idxd.at[slot, 0]], abuf.at[slot],
                             sem_g)
            pltpu.async_copy(b_hbm.at[idxs.at[slot, 0]], bbuf.at[slot],
                             sem_g)

        def wait_gather(slot):
            pltpu.make_async_copy(a_hbm.at[pl.ds(0, _G)], abuf.at[slot],
                                  sem_g).wait()
            pltpu.make_async_copy(b_hbm.at[pl.ds(0, _G)], bbuf.at[slot],
                                  sem_g).wait()

        def wait_write(slot):
            pltpu.make_async_copy(abuf.at[slot], h_hbm.at[pl.ds(0, _G)],
                                  sem_w).wait()

        @pl.when(nv >= 1)
        def _():
            issue_idx(r0, 0)
            wait_idx(0)
            issue_gather(0)

            @pl.when(nv >= 2)
            def _():
                issue_idx(r0 + 1, 1)

        def body(i, carry):
            row = r0 + i
            p = jnp.bitwise_and(i, 1)

            @pl.when(row < R)
            def _():
                wait_gather(p)

                @pl.when((i <= per_w - 3) & (row + 2 < R))
                def _():
                    issue_idx(row + 2, p)

                @pl.when((i <= per_w - 2) & (row + 1 < R))
                def _():
                    @pl.when(i >= 1)
                    def _():
                        wait_write(1 - p)

                    wait_idx(1 - p)
                    issue_gather(1 - p)

                def vbody(r, carry2):
                    for j in range(H // 16):
                        sl = pl.ds(j * 16, 16)
                        abuf[p, r, sl] = jnp.maximum(
                            abuf[p, r, sl] + bbuf[p, r, sl], 0.0)
                    return carry2

                lax.fori_loop(0, _G, vbody, 0)
                pltpu.async_copy(abuf.at[p], h_hbm.at[pl.ds(row * _G, _G)],
                                 sem_w)

            return carry

        lax.fori_loop(0, per_w, body, 0)

        @pl.when(nv >= 2)
        def _():
            wait_write(0)

        @pl.when(nv >= 1)
        def _():
            wait_write(0)

    return gather_k


@functools.lru_cache(maxsize=None)
def _make_scatter(N, R, C, with_counts):
    mesh = plsc.VectorSubcoreMesh(core_axis_name="c", subcore_axis_name="s")
    r_core = R // _NC
    per_s = (r_core + _NS - 1) // _NS
    n_sub = N // _NS
    zrows = 125
    nz = n_sub // zrows
    assert n_sub % zrows == 0

    out_type = [jax.ShapeDtypeStruct((_NC, N, C), jnp.float32)]
    scratch = [
        pltpu.VMEM((2, 1, _G), jnp.int32),
        pltpu.VMEM((2, _G, C), jnp.float32),
        pltpu.VMEM_SHARED((N, C), jnp.float32),
        pltpu.SemaphoreType.DMA,
        pltpu.SemaphoreType.DMA,
    ]
    if with_counts:
        out_type.append(jax.ShapeDtypeStruct((_NC, N, 16), jnp.float32))
        scratch += [
            pltpu.VMEM((_G, 16), jnp.float32),
            pltpu.VMEM_SHARED((N, 16), jnp.float32),
        ]

    @functools.partial(pl.kernel, out_type=out_type, mesh=mesh,
                       scratch_types=scratch, compiler_params=_SC_PARAMS)
    def scatter_k(m_hbm, dst_hbm, zs_hbm, zc_hbm, *refs):
        if with_counts:
            (s_out, c_out, idxd, mbuf, s_sh, sem_idx, sem_m, ones,
             c_sh) = refs
        else:
            s_out, idxd, mbuf, s_sh, sem_idx, sem_m = refs
        c = lax.axis_index("c")
        s = lax.axis_index("s")

        if with_counts:
            def obody(i, carry):
                lane = lax.iota(jnp.int32, 16)
                ones[i, pl.ds(0, 16)] = jnp.where(lane == 0, 1.0, 0.0)
                return carry

            lax.fori_loop(0, _G, obody, 0)
        for k in range(nz):
            dst_sl = pl.ds(s * n_sub + k * zrows, zrows)
            pltpu.sync_copy(zs_hbm, s_sh.at[dst_sl])
            if with_counts:
                pltpu.sync_copy(zc_hbm, c_sh.at[dst_sl])
        plsc.subcore_barrier()

        r0 = c * r_core + s * per_s
        rend = (c + 1) * r_core

        def issue(row, slot):
            pltpu.async_copy(dst_hbm.at[pl.ds(row, 1)], idxd.at[slot],
                             sem_idx)
            pltpu.async_copy(m_hbm.at[pl.ds(row * _G, _G)], mbuf.at[slot],
                             sem_m)

        def wait_in(slot):
            pltpu.make_async_copy(dst_hbm.at[pl.ds(0, 1)], idxd.at[slot],
                                  sem_idx).wait()
            pltpu.make_async_copy(m_hbm.at[pl.ds(0, _G)], mbuf.at[slot],
                                  sem_m).wait()

        @pl.when(r0 < rend)
        def _():
            issue(r0, 0)

        def body(i, carry):
            row = r0 + i
            p = jnp.bitwise_and(i, 1)

            @pl.when(row < rend)
            def _():
                wait_in(p)

                @pl.when((i < per_s - 1) & (row + 1 < rend))
                def _():
                    issue(row + 1, 1 - p)

                pltpu.sync_copy(mbuf.at[p], s_sh.at[idxd.at[p, 0]],
                                add=True)
                if with_counts:
                    pltpu.sync_copy(ones, c_sh.at[idxd.at[p, 0]], add=True)

            return carry

        lax.fori_loop(0, per_s, body, 0)
        plsc.subcore_barrier()

        my = pl.ds(s * n_sub, n_sub)
        pltpu.sync_copy(s_sh.at[my], s_out.at[c, my])
        if with_counts:
            pltpu.sync_copy(c_sh.at[my], c_out.at[c, my])

    return scatter_k


def _dense_pre(x, W1, b1):
    N, D = x.shape
    H = W1.shape[1]

    def body(x_ref, wt_ref, wb_ref, b1_ref, a_ref, b_ref):
        xv = x_ref[...]
        a_ref[...] = (jnp.dot(xv, wt_ref[...],
                              preferred_element_type=jnp.float32)
                      + b1_ref[...])
        b_ref[...] = jnp.dot(xv, wb_ref[...],
                             preferred_element_type=jnp.float32)

    return pl.pallas_call(
        body,
        out_shape=(jax.ShapeDtypeStruct((N, H), jnp.float32),
                   jax.ShapeDtypeStruct((N, H), jnp.float32)),
    )(x, W1[:D], W1[D:], b1.reshape(1, H))


def _combine_pre(S, Cnt, W1, b1):
    _, N, D = S.shape
    H = W1.shape[1]

    def body(s_ref, c_ref, wt_ref, wb_ref, b1_ref, a_ref, b_ref):
        cnt = c_ref[0, :, 0] + c_ref[1, :, 0]
        xv = (s_ref[0] + s_ref[1]) / jnp.maximum(cnt, 1.0)[:, None]
        a_ref[...] = (jnp.dot(xv, wt_ref[...],
                              preferred_element_type=jnp.float32)
                      + b1_ref[...])
        b_ref[...] = jnp.dot(xv, wb_ref[...],
                             preferred_element_type=jnp.float32)

    return pl.pallas_call(
        body,
        out_shape=(jax.ShapeDtypeStruct((N, H), jnp.float32),
                   jax.ShapeDtypeStruct((N, H), jnp.float32)),
    )(S, Cnt, W1[:D], W1[D:], b1.reshape(1, H))


def _edge_mlp(Hm, W2, b2):
    E, H = Hm.shape
    C = W2.shape[1]
    BE = 2560
    assert E % BE == 0

    def body(h_ref, w_ref, b_ref, m_ref):
        m_ref[...] = jnp.maximum(
            jnp.dot(h_ref[...], w_ref[...],
                    preferred_element_type=jnp.float32) + b_ref[...], 0.0)

    return pl.pallas_call(
        body,
        grid=(E // BE,),
        in_specs=[
            pl.BlockSpec((BE, H), lambda i: (i, 0)),
            pl.BlockSpec((H, C), lambda i: (0, 0)),
            pl.BlockSpec((1, C), lambda i: (0, 0)),
        ],
        out_specs=pl.BlockSpec((BE, C), lambda i: (i, 0)),
        out_shape=jax.ShapeDtypeStruct((E, C), jnp.float32),
    )(Hm, W2, b2.reshape(1, C))


def _final_combine(S, Cnt, N):
    _, _, C = S.shape

    def body(s_ref, c_ref, o_ref):
        cnt = c_ref[0, :N, 0] + c_ref[1, :N, 0]
        o_ref[...] = (s_ref[0, :N] + s_ref[1, :N]) / jnp.maximum(
            cnt, 1.0)[:, None]

    return pl.pallas_call(
        body,
        out_shape=jax.ShapeDtypeStruct((N, C), jnp.float32),
    )(S, Cnt)


def kernel(x, edge_index, W1_0, b1_0, W2_0, b2_0, W1_1, b1_1, W2_1, b2_1):
    N, D = x.shape
    E = edge_index.shape[1]
    H = W1_0.shape[1]
    assert E % _G == 0
    R = E // _G

    dst2 = edge_index[0].reshape(R, _G)
    src2 = edge_index[1].reshape(R, _G)

    A0, B0 = _dense_pre(x, W1_0, b1_0)
    Hm0 = _make_gather(N, R, H)(A0, B0, dst2, src2)
    M0 = _edge_mlp(Hm0, W2_0, b2_0)
    zs = jnp.zeros((125, W2_0.shape[1]), jnp.float32)
    zc = jnp.zeros((125, 16), jnp.float32)
    S0, Cnt = _make_scatter(N, R, W2_0.shape[1], True)(M0, dst2, zs, zc)

    A1, B1 = _combine_pre(S0, Cnt, W1_1, b1_1)
    Hm1 = _make_gather(N, R, H)(A1, B1, dst2, src2)
    M1 = _edge_mlp(Hm1, W2_1, b2_1)
    (S1,) = _make_scatter(N, R, W2_1.shape[1], False)(M1, dst2, zs, zc)

    return _final_combine(S1, Cnt, N)


# speedup vs baseline: 3.7138x; 1.0683x over previous
import functools

import jax
import jax.numpy as jnp
from jax import lax
from jax.experimental import pallas as pl
from jax.experimental.pallas import tpu as pltpu
from jax.experimental.pallas import tpu_sc as plsc

_NC = 2
_NS = 16
_NW = _NC * _NS
_G = 128

_SC_PARAMS = pltpu.CompilerParams(use_tc_tiling_on_sc=False)


@functools.lru_cache(maxsize=None)
def _make_gather(N, R, H):
    mesh = plsc.VectorSubcoreMesh(core_axis_name="c", subcore_axis_name="s")
    per_w = (R + _NW - 1) // _NW

    @functools.partial(
        pl.kernel,
        out_type=jax.ShapeDtypeStruct((R * _G, H), jnp.bfloat16),
        mesh=mesh,
        scratch_types=[
            pltpu.VMEM((2, 1, _G), jnp.int32),
            pltpu.VMEM((2, 1, _G), jnp.int32),
            pltpu.VMEM((2, _G, H), jnp.bfloat16),
            pltpu.VMEM((2, _G, H), jnp.bfloat16),
            pltpu.SemaphoreType.DMA,
            pltpu.SemaphoreType.DMA,
            pltpu.SemaphoreType.DMA,
        ],
        compiler_params=_SC_PARAMS,
    )
    def gather_k(a_hbm, b_hbm, dst_hbm, src_hbm, h_hbm, idxd, idxs, abuf,
                 bbuf, sem_idx, sem_g, sem_w):
        c = lax.axis_index("c")
        s = lax.axis_index("s")
        wid = s * _NC + c
        r0 = wid * per_w
        nv = jnp.clip(R - r0, 0, per_w)

        def issue_idx(row, slot):
            pltpu.async_copy(dst_hbm.at[pl.ds(row, 1)], idxd.at[slot],
                             sem_idx)
            pltpu.async_copy(src_hbm.at[pl.ds(row, 1)], idxs.at[slot],
                             sem_idx)

        def wait_idx(slot):
            pltpu.make_async_copy(dst_hbm.at[pl.ds(0, 1)], idxd.at[slot],
                                  sem_idx).wait()
            pltpu.make_async_copy(src_hbm.at[pl.ds(0, 1)], idxs.at[slot],
                                  sem_idx).wait()

        def issue_gather(slot):
            for h0 in (0, 64):
                sl = pl.ds(h0, 64)
                pltpu.async_copy(a_hbm.at[idxd.at[slot, 0, sl]],
                                 abuf.at[slot, sl], sem_g)
                pltpu.async_copy(b_hbm.at[idxs.at[slot, 0, sl]],
                                 bbuf.at[slot, sl], sem_g)

        def wait_gather(slot):
            for _ in range(2):
                pltpu.make_async_copy(a_hbm.at[pl.ds(0, 64)],
                                      abuf.at[slot, pl.ds(0, 64)],
                                      sem_g).wait()
                pltpu.make_async_copy(b_hbm.at[pl.ds(0, 64)],
                                      bbuf.at[slot, pl.ds(0, 64)],
                                      sem_g).wait()

        def wait_write(slot):
            pltpu.make_async_copy(abuf.at[slot], h_hbm.at[pl.ds(0, _G)],
                                  sem_w).wait()

        @pl.when(nv >= 1)
        def _():
            issue_idx(r0, 0)
            wait_idx(0)
            issue_gather(0)

            @pl.when(nv >= 2)
            def _():
                issue_idx(r0 + 1, 1)

        def body(i, carry):
            row = r0 + i
            p = jnp.bitwise_and(i, 1)

            @pl.when(row < R)
            def _():
                wait_gather(p)

                @pl.when((i <= per_w - 3) & (row + 2 < R))
                def _():
                    issue_idx(row + 2, p)

                @pl.when((i <= per_w - 2) & (row + 1 < R))
                def _():
                    @pl.when(i >= 1)
                    def _():
                        wait_write(1 - p)

                    wait_idx(1 - p)
                    issue_gather(1 - p)

                def vbody(r, carry2):
                    for j in range(H // 32):
                        sl = pl.ds(j * 32, 32)
                        abuf[p, r, sl] = jnp.maximum(
                            abuf[p, r, sl] + bbuf[p, r, sl],
                            jnp.bfloat16(0.0))
                    return carry2

                lax.fori_loop(0, _G, vbody, 0)
                pltpu.async_copy(abuf.at[p], h_hbm.at[pl.ds(row * _G, _G)],
                                 sem_w)

            return carry

        lax.fori_loop(0, per_w, body, 0)

        @pl.when(nv >= 2)
        def _():
            wait_write(0)

        @pl.when(nv >= 1)
        def _():
            wait_write(0)

    return gather_k


@functools.lru_cache(maxsize=None)
def _make_scatter(N, R, C, with_counts):
    mesh = plsc.VectorSubcoreMesh(core_axis_name="c", subcore_axis_name="s")
    r_core = R // _NC
    per_s = (r_core + _NS - 1) // _NS
    n_sub = N // _NS
    zrows = 125
    nz = n_sub // zrows
    assert n_sub % zrows == 0

    out_type = [jax.ShapeDtypeStruct((_NC, N, C), jnp.float32)]
    scratch = [
        pltpu.VMEM((2, 1, _G), jnp.int32),
        pltpu.VMEM((2, _G, C), jnp.float32),
        pltpu.VMEM_SHARED((N, C), jnp.float32),
        pltpu.SemaphoreType.DMA,
        pltpu.SemaphoreType.DMA,
    ]
    if with_counts:
        out_type.append(jax.ShapeDtypeStruct((_NC, N, 16), jnp.float32))
        scratch += [
            pltpu.VMEM((_G, 16), jnp.float32),
            pltpu.VMEM_SHARED((N, 16), jnp.float32),
        ]

    @functools.partial(pl.kernel, out_type=out_type, mesh=mesh,
                       scratch_types=scratch, compiler_params=_SC_PARAMS)
    def scatter_k(m_hbm, dst_hbm, zs_hbm, zc_hbm, *refs):
        if with_counts:
            (s_out, c_out, idxd, mbuf, s_sh, sem_idx, sem_m, ones,
             c_sh) = refs
        else:
            s_out, idxd, mbuf, s_sh, sem_idx, sem_m = refs
        c = lax.axis_index("c")
        s = lax.axis_index("s")

        if with_counts:
            def obody(i, carry):
                lane = lax.iota(jnp.int32, 16)
                ones[i, pl.ds(0, 16)] = jnp.where(lane == 0, 1.0, 0.0)
                return carry

            lax.fori_loop(0, _G, obody, 0)
        for k in range(nz):
            dst_sl = pl.ds(s * n_sub + k * zrows, zrows)
            pltpu.sync_copy(zs_hbm, s_sh.at[dst_sl])
            if with_counts:
                pltpu.sync_copy(zc_hbm, c_sh.at[dst_sl])
        plsc.subcore_barrier()

        r0 = c * r_core + s * per_s
        rend = (c + 1) * r_core

        def issue(row, slot):
            pltpu.async_copy(dst_hbm.at[pl.ds(row, 1)], idxd.at[slot],
                             sem_idx)
            pltpu.async_copy(m_hbm.at[pl.ds(row * _G, _G)], mbuf.at[slot],
                             sem_m)

        def wait_in(slot):
            pltpu.make_async_copy(dst_hbm.at[pl.ds(0, 1)], idxd.at[slot],
                                  sem_idx).wait()
            pltpu.make_async_copy(m_hbm.at[pl.ds(0, _G)], mbuf.at[slot],
                                  sem_m).wait()

        @pl.when(r0 < rend)
        def _():
            issue(r0, 0)

        def body(i, carry):
            row = r0 + i
            p = jnp.bitwise_and(i, 1)

            @pl.when(row < rend)
            def _():
                wait_in(p)

                @pl.when((i < per_s - 1) & (row + 1 < rend))
                def _():
                    issue(row + 1, 1 - p)

                pltpu.sync_copy(mbuf.at[p], s_sh.at[idxd.at[p, 0]],
                                add=True)
                if with_counts:
                    pltpu.sync_copy(ones, c_sh.at[idxd.at[p, 0]], add=True)

            return carry

        lax.fori_loop(0, per_s, body, 0)
        plsc.subcore_barrier()

        my = pl.ds(s * n_sub, n_sub)
        pltpu.sync_copy(s_sh.at[my], s_out.at[c, my])
        if with_counts:
            pltpu.sync_copy(c_sh.at[my], c_out.at[c, my])

    return scatter_k


def _dense_pre(x, W1, b1):
    N, D = x.shape
    H = W1.shape[1]

    def body(x_ref, wt_ref, wb_ref, b1_ref, a_ref, b_ref):
        xv = x_ref[...]
        a_ref[...] = (jnp.dot(xv, wt_ref[...],
                              preferred_element_type=jnp.float32)
                      + b1_ref[...]).astype(jnp.bfloat16)
        b_ref[...] = jnp.dot(xv, wb_ref[...],
                             preferred_element_type=jnp.float32
                             ).astype(jnp.bfloat16)

    return pl.pallas_call(
        body,
        out_shape=(jax.ShapeDtypeStruct((N, H), jnp.bfloat16),
                   jax.ShapeDtypeStruct((N, H), jnp.bfloat16)),
    )(x, W1[:D], W1[D:], b1.reshape(1, H))


def _combine_pre(S, Cnt, W1, b1):
    _, N, D = S.shape
    H = W1.shape[1]

    def body(s_ref, c_ref, wt_ref, wb_ref, b1_ref, a_ref, b_ref):
        cnt = c_ref[0, :, 0] + c_ref[1, :, 0]
        xv = (s_ref[0] + s_ref[1]) / jnp.maximum(cnt, 1.0)[:, None]
        a_ref[...] = (jnp.dot(xv, wt_ref[...],
                              preferred_element_type=jnp.float32)
                      + b1_ref[...]).astype(jnp.bfloat16)
        b_ref[...] = jnp.dot(xv, wb_ref[...],
                             preferred_element_type=jnp.float32
                             ).astype(jnp.bfloat16)

    return pl.pallas_call(
        body,
        out_shape=(jax.ShapeDtypeStruct((N, H), jnp.bfloat16),
                   jax.ShapeDtypeStruct((N, H), jnp.bfloat16)),
    )(S, Cnt, W1[:D], W1[D:], b1.reshape(1, H))


def _edge_mlp(Hm, W2, b2):
    E, H = Hm.shape
    C = W2.shape[1]
    W2 = W2.astype(jnp.bfloat16)
    BE = 2560
    assert E % BE == 0

    def body(h_ref, w_ref, b_ref, m_ref):
        m_ref[...] = jnp.maximum(
            jnp.dot(h_ref[...], w_ref[...],
                    preferred_element_type=jnp.float32) + b_ref[...], 0.0)

    return pl.pallas_call(
        body,
        grid=(E // BE,),
        in_specs=[
            pl.BlockSpec((BE, H), lambda i: (i, 0)),
            pl.BlockSpec((H, C), lambda i: (0, 0)),
            pl.BlockSpec((1, C), lambda i: (0, 0)),
        ],
        out_specs=pl.BlockSpec((BE, C), lambda i: (i, 0)),
        out_shape=jax.ShapeDtypeStruct((E, C), jnp.float32),
    )(Hm, W2, b2.reshape(1, C))


def _final_combine(S, Cnt, N):
    _, _, C = S.shape

    def body(s_ref, c_ref, o_ref):
        cnt = c_ref[0, :N, 0] + c_ref[1, :N, 0]
        o_ref[...] = (s_ref[0, :N] + s_ref[1, :N]) / jnp.maximum(
            cnt, 1.0)[:, None]

    return pl.pallas_call(
        body,
        out_shape=jax.ShapeDtypeStruct((N, C), jnp.float32),
    )(S, Cnt)


def kernel(x, edge_index, W1_0, b1_0, W2_0, b2_0, W1_1, b1_1, W2_1, b2_1):
    N, D = x.shape
    E = edge_index.shape[1]
    H = W1_0.shape[1]
    assert E % _G == 0
    R = E // _G

    dst2 = edge_index[0].reshape(R, _G)
    src2 = edge_index[1].reshape(R, _G)

    A0, B0 = _dense_pre(x, W1_0, b1_0)
    Hm0 = _make_gather(N, R, H)(A0, B0, dst2, src2)
    M0 = _edge_mlp(Hm0, W2_0, b2_0)
    zs = jnp.zeros((125, W2_0.shape[1]), jnp.float32)
    zc = jnp.zeros((125, 16), jnp.float32)
    S0, Cnt = _make_scatter(N, R, W2_0.shape[1], True)(M0, dst2, zs, zc)

    A1, B1 = _combine_pre(S0, Cnt, W1_1, b1_1)
    Hm1 = _make_gather(N, R, H)(A1, B1, dst2, src2)
    M1 = _edge_mlp(Hm1, W2_1, b2_1)
    (S1,) = _make_scatter(N, R, W2_1.shape[1], False)(M1, dst2, zs, zc)

    return _final_combine(S1, Cnt, N)


# speedup vs baseline: 4.5861x; 1.2349x over previous
import functools

import jax
import jax.numpy as jnp
from jax import lax
from jax.experimental import pallas as pl
from jax.experimental.pallas import tpu as pltpu
from jax.experimental.pallas import tpu_sc as plsc

_NC = 2
_NS = 16
_NW = _NC * _NS
_G = 128

_SC_PARAMS = pltpu.CompilerParams(use_tc_tiling_on_sc=False,
                                  needs_layout_passes=False)


@functools.lru_cache(maxsize=None)
def _make_gather(N, R, H):
    mesh = plsc.VectorSubcoreMesh(core_axis_name="c", subcore_axis_name="s")
    per_w = (R + _NW - 1) // _NW

    @functools.partial(
        pl.kernel,
        out_type=jax.ShapeDtypeStruct((R * _G // 2, 2 * H), jnp.float32),
        mesh=mesh,
        scratch_types=[
            pltpu.VMEM((2, 1, _G), jnp.int32),
            pltpu.VMEM((2, 1, _G), jnp.int32),
            pltpu.VMEM((2, _G, H), jnp.bfloat16),
            pltpu.VMEM((2, _G, H), jnp.bfloat16),
            pltpu.VMEM((2, _G // 2, 2 * H), jnp.float32),
            pltpu.SemaphoreType.DMA,
            pltpu.SemaphoreType.DMA,
            pltpu.SemaphoreType.DMA,
        ],
        compiler_params=_SC_PARAMS,
    )
    def gather_k(a_hbm, b_hbm, dst_hbm, src_hbm, h_hbm, idxd, idxs, abuf,
                 bbuf, obuf, sem_idx, sem_g, sem_w):
        c = lax.axis_index("c")
        s = lax.axis_index("s")
        wid = s * _NC + c
        r0 = wid * per_w
        nv = jnp.clip(R - r0, 0, per_w)

        def issue_idx(row, slot):
            pltpu.async_copy(dst_hbm.at[pl.ds(row, 1)], idxd.at[slot],
                             sem_idx)
            pltpu.async_copy(src_hbm.at[pl.ds(row, 1)], idxs.at[slot],
                             sem_idx)

        def wait_idx(slot):
            pltpu.make_async_copy(dst_hbm.at[pl.ds(0, 1)], idxd.at[slot],
                                  sem_idx).wait()
            pltpu.make_async_copy(src_hbm.at[pl.ds(0, 1)], idxs.at[slot],
                                  sem_idx).wait()

        def issue_gather(slot):
            for h0 in (0, 64):
                sl = pl.ds(h0, 64)
                pltpu.async_copy(a_hbm.at[idxd.at[slot, 0, sl]],
                                 abuf.at[slot, sl], sem_g)
                pltpu.async_copy(b_hbm.at[idxs.at[slot, 0, sl]],
                                 bbuf.at[slot, sl], sem_g)

        def wait_gather(slot):
            for _ in range(2):
                pltpu.make_async_copy(a_hbm.at[pl.ds(0, 64)],
                                      abuf.at[slot, pl.ds(0, 64)],
                                      sem_g).wait()
                pltpu.make_async_copy(b_hbm.at[pl.ds(0, 64)],
                                      bbuf.at[slot, pl.ds(0, 64)],
                                      sem_g).wait()

        def wait_write(slot):
            pltpu.make_async_copy(obuf.at[slot],
                                  h_hbm.at[pl.ds(0, _G // 2)],
                                  sem_w).wait()

        @pl.when(nv >= 1)
        def _():
            issue_idx(r0, 0)
            wait_idx(0)
            issue_gather(0)

            @pl.when(nv >= 2)
            def _():
                issue_idx(r0 + 1, 1)

        def body(i, carry):
            row = r0 + i
            p = jnp.bitwise_and(i, 1)

            @pl.when(row < R)
            def _():
                wait_gather(p)

                @pl.when((i <= per_w - 3) & (row + 2 < R))
                def _():
                    issue_idx(row + 2, p)

                @pl.when((i <= per_w - 2) & (row + 1 < R))
                def _():
                    @pl.when(i >= 1)
                    def _():
                        wait_write(1 - p)

                    wait_idx(1 - p)
                    issue_gather(1 - p)

                msk = jnp.uint32(0xFFFF0000)

                def vbody(r, carry2):
                    q = lax.shift_right_logical(r, 1)
                    off = jnp.bitwise_and(r, 1) * (2 * H // 2)
                    for j in range(H // 32):
                        sl = pl.ds(j * 32, 32)
                        hv = jnp.maximum(abuf[p, r, sl] + bbuf[p, r, sl],
                                         jnp.bfloat16(0.0))
                        u = plsc.bitcast(hv, jnp.uint32)
                        lo = plsc.bitcast(u << 16, jnp.float32)
                        hi = plsc.bitcast(jnp.bitwise_and(u, msk),
                                          jnp.float32)
                        obuf[p, q, pl.dslice(off + 32 * j, 16)] = lo
                        obuf[p, q, pl.dslice(off + 32 * j + 16, 16)] = hi
                    return carry2

                lax.fori_loop(0, _G, vbody, 0)
                pltpu.async_copy(
                    obuf.at[p],
                    h_hbm.at[pl.ds(row * (_G // 2), _G // 2)], sem_w)

            return carry

        lax.fori_loop(0, per_w, body, 0)

        @pl.when(nv >= 2)
        def _():
            wait_write(0)

        @pl.when(nv >= 1)
        def _():
            wait_write(0)

    return gather_k


@functools.lru_cache(maxsize=None)
def _make_scatter(N, R, C, with_counts):
    mesh = plsc.VectorSubcoreMesh(core_axis_name="c", subcore_axis_name="s")
    r_core = R // _NC
    per_s = (r_core + _NS - 1) // _NS
    n_sub = N // _NS
    zrows = 125
    nz = n_sub // zrows
    assert n_sub % zrows == 0

    out_type = [jax.ShapeDtypeStruct((_NC, N, C), jnp.float32)]
    scratch = [
        pltpu.VMEM((2, 1, _G), jnp.int32),
        pltpu.VMEM((2, _G, C), jnp.float32),
        pltpu.VMEM_SHARED((N, C), jnp.float32),
        pltpu.SemaphoreType.DMA,
        pltpu.SemaphoreType.DMA,
    ]
    if with_counts:
        out_type.append(jax.ShapeDtypeStruct((_NC, N, 16), jnp.float32))
        scratch += [
            pltpu.VMEM((_G, 16), jnp.float32),
            pltpu.VMEM_SHARED((N, 16), jnp.float32),
        ]

    @functools.partial(pl.kernel, out_type=out_type, mesh=mesh,
                       scratch_types=scratch, compiler_params=_SC_PARAMS)
    def scatter_k(m_hbm, dst_hbm, zs_hbm, zc_hbm, *refs):
        if with_counts:
            (s_out, c_out, idxd, mbuf, s_sh, sem_idx, sem_m, ones,
             c_sh) = refs
        else:
            s_out, idxd, mbuf, s_sh, sem_idx, sem_m = refs
        c = lax.axis_index("c")
        s = lax.axis_index("s")

        if with_counts:
            def obody(i, carry):
                lane = lax.iota(jnp.int32, 16)
                ones[i, pl.ds(0, 16)] = jnp.where(lane == 0, 1.0, 0.0)
                return carry

            lax.fori_loop(0, _G, obody, 0)
        for k in range(nz):
            dst_sl = pl.ds(s * n_sub + k * zrows, zrows)
            pltpu.sync_copy(zs_hbm, s_sh.at[dst_sl])
            if with_counts:
                pltpu.sync_copy(zc_hbm, c_sh.at[dst_sl])
        plsc.subcore_barrier()

        r0 = c * r_core + s * per_s
        rend = (c + 1) * r_core

        def issue(row, slot):
            pltpu.async_copy(dst_hbm.at[pl.ds(row, 1)], idxd.at[slot],
                             sem_idx)
            pltpu.async_copy(m_hbm.at[pl.ds(row * _G, _G)], mbuf.at[slot],
                             sem_m)

        def wait_in(slot):
            pltpu.make_async_copy(dst_hbm.at[pl.ds(0, 1)], idxd.at[slot],
                                  sem_idx).wait()
            pltpu.make_async_copy(m_hbm.at[pl.ds(0, _G)], mbuf.at[slot],
                                  sem_m).wait()

        @pl.when(r0 < rend)
        def _():
            issue(r0, 0)

        def body(i, carry):
            row = r0 + i
            p = jnp.bitwise_and(i, 1)

            @pl.when(row < rend)
            def _():
                wait_in(p)

                @pl.when((i < per_s - 1) & (row + 1 < rend))
                def _():
                    issue(row + 1, 1 - p)

                pltpu.sync_copy(mbuf.at[p], s_sh.at[idxd.at[p, 0]],
                                add=True)
                if with_counts:
                    pltpu.sync_copy(ones, c_sh.at[idxd.at[p, 0]], add=True)

            return carry

        lax.fori_loop(0, per_s, body, 0)
        plsc.subcore_barrier()

        my = pl.ds(s * n_sub, n_sub)
        pltpu.sync_copy(s_sh.at[my], s_out.at[c, my])
        if with_counts:
            pltpu.sync_copy(c_sh.at[my], c_out.at[c, my])

    return scatter_k


def _dense_pre(x, W1, b1):
    N, D = x.shape
    H = W1.shape[1]

    def body(x_ref, wt_ref, wb_ref, b1_ref, a_ref, b_ref):
        xv = x_ref[...]
        a_ref[...] = (jnp.dot(xv, wt_ref[...],
                              preferred_element_type=jnp.float32)
                      + b1_ref[...]).astype(jnp.bfloat16)
        b_ref[...] = jnp.dot(xv, wb_ref[...],
                             preferred_element_type=jnp.float32
                             ).astype(jnp.bfloat16)

    return pl.pallas_call(
        body,
        out_shape=(jax.ShapeDtypeStruct((N, H), jnp.bfloat16),
                   jax.ShapeDtypeStruct((N, H), jnp.bfloat16)),
    )(x, W1[:D], W1[D:], b1.reshape(1, H))


def _combine_pre(S, Cnt, W1, b1):
    _, N, D = S.shape
    H = W1.shape[1]

    def body(s_ref, c_ref, wt_ref, wb_ref, b1_ref, a_ref, b_ref):
        cnt = c_ref[0, :, 0] + c_ref[1, :, 0]
        xv = (s_ref[0] + s_ref[1]) / jnp.maximum(cnt, 1.0)[:, None]
        a_ref[...] = (jnp.dot(xv, wt_ref[...],
                              preferred_element_type=jnp.float32)
                      + b1_ref[...]).astype(jnp.bfloat16)
        b_ref[...] = jnp.dot(xv, wb_ref[...],
                             preferred_element_type=jnp.float32
                             ).astype(jnp.bfloat16)

    return pl.pallas_call(
        body,
        out_shape=(jax.ShapeDtypeStruct((N, H), jnp.bfloat16),
                   jax.ShapeDtypeStruct((N, H), jnp.bfloat16)),
    )(S, Cnt, W1[:D], W1[D:], b1.reshape(1, H))


def _edge_mlp(Hm2, W2p, b2, E):
    H = W2p.shape[0]
    C = W2p.shape[1]
    BE = 2560
    BE2 = BE // 2
    assert E % BE == 0

    def body(h_ref, w_ref, b_ref, m_ref):
        h2 = h_ref[...]
        w = w_ref[...]
        b = b_ref[...]
        ml = jnp.maximum(
            jnp.dot(h2[:, :H], w, preferred_element_type=jnp.float32)
            + b, 0.0)
        mr = jnp.maximum(
            jnp.dot(h2[:, H:], w, preferred_element_type=jnp.float32)
            + b, 0.0)
        m_ref[...] = jnp.concatenate(
            [ml[:, None, :], mr[:, None, :]], axis=1).reshape(BE, C)

    return pl.pallas_call(
        body,
        grid=(E // BE,),
        in_specs=[
            pl.BlockSpec((BE2, 2 * H), lambda i: (i, 0)),
            pl.BlockSpec((H, C), lambda i: (0, 0)),
            pl.BlockSpec((1, C), lambda i: (0, 0)),
        ],
        out_specs=pl.BlockSpec((BE, C), lambda i: (i, 0)),
        out_shape=jax.ShapeDtypeStruct((E, C), jnp.float32),
    )(Hm2, W2p, b2.reshape(1, C))


def _final_combine(S, Cnt, N):
    _, _, C = S.shape

    def body(s_ref, c_ref, o_ref):
        cnt = c_ref[0, :N, 0] + c_ref[1, :N, 0]
        o_ref[...] = (s_ref[0, :N] + s_ref[1, :N]) / jnp.maximum(
            cnt, 1.0)[:, None]

    return pl.pallas_call(
        body,
        out_shape=jax.ShapeDtypeStruct((N, C), jnp.float32),
    )(S, Cnt)


def kernel(x, edge_index, W1_0, b1_0, W2_0, b2_0, W1_1, b1_1, W2_1, b2_1):
    N, D = x.shape
    E = edge_index.shape[1]
    H = W1_0.shape[1]
    assert E % _G == 0
    R = E // _G

    dst2 = edge_index[0].reshape(R, _G)
    src2 = edge_index[1].reshape(R, _G)

    pos = jnp.arange(H)
    perm = 32 * (pos // 32) + 2 * (pos % 16) + ((pos % 32) // 16)
    W2p_0 = W2_0[perm]
    W2p_1 = W2_1[perm]

    A0, B0 = _dense_pre(x, W1_0, b1_0)
    Hm0 = _make_gather(N, R, H)(A0, B0, dst2, src2)
    M0 = _edge_mlp(Hm0, W2p_0, b2_0, E)
    zs = jnp.zeros((125, W2_0.shape[1]), jnp.float32)
    zc = jnp.zeros((125, 16), jnp.float32)
    S0, Cnt = _make_scatter(N, R, W2_0.shape[1], True)(M0, dst2, zs, zc)

    A1, B1 = _combine_pre(S0, Cnt, W1_1, b1_1)
    Hm1 = _make_gather(N, R, H)(A1, B1, dst2, src2)
    M1 = _edge_mlp(Hm1, W2p_1, b2_1, E)
    (S1,) = _make_scatter(N, R, W2_1.shape[1], False)(M1, dst2, zs, zc)

    return _final_combine(S1, Cnt, N)
